# baseline (speedup 1.0000x reference)
import functools

import jax
import jax.numpy as jnp
from jax import lax
from jax.experimental import pallas as pl
from jax.experimental.pallas import tpu as pltpu
from jax.experimental.pallas import tpu_sc as plsc

_NB = 128
_SCALE = 1.0
_N = 262144

_INFO = plsc.get_sparse_core_info()
_NC = _INFO.num_cores
_NS = _INFO.num_subcores
_NW = _NC * _NS
_L = _INFO.num_lanes
_PPW = _N // _NW
_CHUNK = 128
_NCHUNK = _PPW // _CHUNK


def _body(x_hbm, y_hbm, z_hbm, table_hbm,
          o0_hbm, o1_hbm, o2_hbm, o3_hbm,
          x_v, y_v, z_v, cond_v, e0_v, e1_v, e2_v, e3_v,
          c0_v, c1_v, c2_v, c3_v, sem):
    wid = lax.axis_index("s") * _NC + lax.axis_index("c")
    base = wid * _PPW

    with jax.named_scope("stage_xyz"):
        pltpu.sync_copy(x_hbm.at[pl.ds(base, _PPW)], x_v)
        pltpu.sync_copy(y_hbm.at[pl.ds(base, _PPW)], y_v)
        pltpu.sync_copy(z_hbm.at[pl.ds(base, _PPW)], z_v)

    inv_cell = jnp.float32(_NB / _SCALE)
    half_nb = jnp.float32(_NB // 2)
    bound = jnp.float32(_SCALE / 2)

    def phase1(i, _):
        s = i * _L
        x = x_v[pl.ds(s, _L)]
        y = y_v[pl.ds(s, _L)]
        z = z_v[pl.ds(s, _L)]
        ix = jnp.clip((x * inv_cell + half_nb).astype(jnp.int32), 0, _NB - 1)
        iy = jnp.clip((y * inv_cell + half_nb).astype(jnp.int32), 0, _NB - 1)
        iz = jnp.clip((z * inv_cell + half_nb).astype(jnp.int32), 0, _NB - 1)
        e0 = ((ix * _NB + iy) << 9) + iz
        e0_v[pl.ds(s, _L)] = e0
        e1_v[pl.ds(s, _L)] = e0 + 128
        e2_v[pl.ds(s, _L)] = e0 + 256
        e3_v[pl.ds(s, _L)] = e0 + 384
        cond = (jnp.abs(x) < bound) & (jnp.abs(y) < bound) & (jnp.abs(z) < bound)
        cond_v[pl.ds(s, _L)] = jnp.where(cond, 1.0, 0.0).astype(jnp.float32)
        return 0

    with jax.named_scope("phase1"):
        lax.fori_loop(0, _PPW // _L, phase1, 0)

    evs = (e0_v, e1_v, e2_v, e3_v)
    cvs = (c0_v, c1_v, c2_v, c3_v)

    def mk(ev, cv):
        def fire(c, _c):
            pltpu.make_async_copy(
                table_hbm.at[ev.at[pl.ds(c * _CHUNK, _CHUNK)]],
                cv.at[pl.ds(c * _CHUNK, _CHUNK)],
                sem,
            ).start()
            return 0

        def drain(c, _c):
            pltpu.make_async_copy(
                table_hbm.at[ev.at[pl.ds(c * _CHUNK, _CHUNK)]],
                cv.at[pl.ds(c * _CHUNK, _CHUNK)],
                sem,
            ).wait()
            return 0

        return fire, drain

    fds = [mk(ev, cv) for ev, cv in zip(evs, cvs)]
    with jax.named_scope("fire"):
        for fire, _ in fds:
            lax.fori_loop(0, _NCHUNK, fire, 0)
    with jax.named_scope("drain"):
        for _, drain in fds:
            lax.fori_loop(0, _NCHUNK, drain, 0)

    def phase3(i, _):
        s = i * _L
        m = cond_v[pl.ds(s, _L)]
        v0 = c0_v[pl.ds(s, _L)] * m
        v1 = c1_v[pl.ds(s, _L)] * m
        v2 = c2_v[pl.ds(s, _L)] * m
        v3 = c3_v[pl.ds(s, _L)] * m
        c0_v[pl.ds(s, _L)] = 1.0 / (1.0 + jnp.exp(-v0))
        c1_v[pl.ds(s, _L)] = 1.0 / (1.0 + jnp.exp(-v1))
        c2_v[pl.ds(s, _L)] = 1.0 / (1.0 + jnp.exp(-v2))
        c3_v[pl.ds(s, _L)] = jnp.maximum(v3, 0.0)
        return 0

    with jax.named_scope("phase3"):
        lax.fori_loop(0, _PPW // _L, phase3, 0)

    with jax.named_scope("out"):
        pltpu.sync_copy(c0_v, o0_hbm.at[pl.ds(base, _PPW)])
        pltpu.sync_copy(c1_v, o1_hbm.at[pl.ds(base, _PPW)])
        pltpu.sync_copy(c2_v, o2_hbm.at[pl.ds(base, _PPW)])
        pltpu.sync_copy(c3_v, o3_hbm.at[pl.ds(base, _PPW)])


@jax.jit
def _run(x, y, z, table):
    mesh = plsc.VectorSubcoreMesh(core_axis_name="c", subcore_axis_name="s")
    f = functools.partial(
        pl.kernel,
        mesh=mesh,
        out_type=[jax.ShapeDtypeStruct((_N,), jnp.float32)] * 4,
        scratch_types=[
            pltpu.VMEM((_PPW,), jnp.float32),
            pltpu.VMEM((_PPW,), jnp.float32),
            pltpu.VMEM((_PPW,), jnp.float32),
            pltpu.VMEM((_PPW,), jnp.float32),
            pltpu.VMEM((_PPW,), jnp.int32),
            pltpu.VMEM((_PPW,), jnp.int32),
            pltpu.VMEM((_PPW,), jnp.int32),
            pltpu.VMEM((_PPW,), jnp.int32),
            pltpu.VMEM((_PPW,), jnp.float32),
            pltpu.VMEM((_PPW,), jnp.float32),
            pltpu.VMEM((_PPW,), jnp.float32),
            pltpu.VMEM((_PPW,), jnp.float32),
            pltpu.SemaphoreType.DMA,
        ],
        compiler_params=pltpu.CompilerParams(
            needs_layout_passes=False, use_tc_tiling_on_sc=False),
    )(_body)
    return f(x, y, z, table)


def kernel(xyz, d, voxels):
    del d
    table = jnp.transpose(voxels, (0, 1, 3, 2)).reshape(-1)
    c0, c1, c2, c3 = _run(xyz[:, 0], xyz[:, 1], xyz[:, 2], table)
    colors = jnp.stack([c0, c1, c2], axis=1)
    return colors, c3


# speedup vs baseline: 1.2952x; 1.2952x over previous
import functools

import jax
import jax.numpy as jnp
from jax import lax
from jax.experimental import pallas as pl
from jax.experimental.pallas import tpu as pltpu
from jax.experimental.pallas import tpu_sc as plsc

_NB = 128
_SCALE = 1.0
_N = 262144

_INFO = plsc.get_sparse_core_info()
_NC = _INFO.num_cores
_NS = _INFO.num_subcores
_NW = _NC * _NS
_L = _INFO.num_lanes
_PPW = _N // _NW
_CHUNK = 128
_NCHUNK = _PPW // _CHUNK
_SUB = _CHUNK // _L
_RING = 8


def _body(x_hbm, y_hbm, z_hbm, table_hbm,
          o0_hbm, o1_hbm, o2_hbm, o3_hbm,
          x_v, y_v, z_v, cond_v, e0_v, e1_v, e2_v, e3_v,
          c0_v, c1_v, c2_v, c3_v, sems):
    wid = lax.axis_index("s") * _NC + lax.axis_index("c")
    base = wid * _PPW

    with jax.named_scope("stage_xyz"):
        pltpu.sync_copy(x_hbm.at[pl.ds(base, _PPW)], x_v)
        pltpu.sync_copy(y_hbm.at[pl.ds(base, _PPW)], y_v)
        pltpu.sync_copy(z_hbm.at[pl.ds(base, _PPW)], z_v)

    inv_cell = jnp.float32(_NB / _SCALE)
    half_nb = jnp.float32(_NB // 2)
    bound = jnp.float32(_SCALE / 2)

    evs = (e0_v, e1_v, e2_v, e3_v)
    cvs = (c0_v, c1_v, c2_v, c3_v)

    def p1(c):
        @plsc.parallel_loop(0, _SUB, unroll=2)
        def _(k):
            s = c * _CHUNK + k * _L
            x = x_v[pl.ds(s, _L)]
            y = y_v[pl.ds(s, _L)]
            z = z_v[pl.ds(s, _L)]
            ix = jnp.clip((x * inv_cell + half_nb).astype(jnp.int32), 0, _NB - 1)
            iy = jnp.clip((y * inv_cell + half_nb).astype(jnp.int32), 0, _NB - 1)
            iz = jnp.clip((z * inv_cell + half_nb).astype(jnp.int32), 0, _NB - 1)
            e0 = ((ix * _NB + iy) << 9) + iz
            e0_v[pl.ds(s, _L)] = e0
            e1_v[pl.ds(s, _L)] = e0 + 128
            e2_v[pl.ds(s, _L)] = e0 + 256
            e3_v[pl.ds(s, _L)] = e0 + 384
            cond = ((jnp.abs(x) < bound) & (jnp.abs(y) < bound)
                    & (jnp.abs(z) < bound))
            cond_v[pl.ds(s, _L)] = jnp.where(cond, 1.0, 0.0).astype(jnp.float32)

    def fire(c):
        sem = sems.at[lax.rem(c, _RING)]
        for ev, cv in zip(evs, cvs):
            pltpu.make_async_copy(
                table_hbm.at[ev.at[pl.ds(c * _CHUNK, _CHUNK)]],
                cv.at[pl.ds(c * _CHUNK, _CHUNK)],
                sem,
            ).start()

    def drain(c):
        sem = sems.at[lax.rem(c, _RING)]
        for ev, cv in zip(evs, cvs):
            pltpu.make_async_copy(
                table_hbm.at[ev.at[pl.ds(c * _CHUNK, _CHUNK)]],
                cv.at[pl.ds(c * _CHUNK, _CHUNK)],
                sem,
            ).wait()

    def p3(c):
        @plsc.parallel_loop(0, _SUB, unroll=2)
        def _(k):
            s = c * _CHUNK + k * _L
            m = cond_v[pl.ds(s, _L)]
            v0 = c0_v[pl.ds(s, _L)] * m
            v1 = c1_v[pl.ds(s, _L)] * m
            v2 = c2_v[pl.ds(s, _L)] * m
            v3 = c3_v[pl.ds(s, _L)] * m
            c0_v[pl.ds(s, _L)] = 1.0 / (1.0 + jnp.exp(-v0))
            c1_v[pl.ds(s, _L)] = 1.0 / (1.0 + jnp.exp(-v1))
            c2_v[pl.ds(s, _L)] = 1.0 / (1.0 + jnp.exp(-v2))
            c3_v[pl.ds(s, _L)] = jnp.maximum(v3, 0.0)

    def prof(c, _):
        p1(c)
        fire(c)
        return 0

    with jax.named_scope("prologue"):
        lax.fori_loop(0, _RING, prof, 0)

    def steady(c, _):
        p1(c + _RING)
        drain(c)
        fire(c + _RING)
        p3(c)
        return 0

    with jax.named_scope("steady"):
        lax.fori_loop(0, _NCHUNK - _RING, steady, 0)

    def epi(c, _):
        drain(c)
        p3(c)
        return 0

    with jax.named_scope("epilogue"):
        lax.fori_loop(_NCHUNK - _RING, _NCHUNK, epi, 0)

    with jax.named_scope("out"):
        pltpu.sync_copy(c0_v, o0_hbm.at[pl.ds(base, _PPW)])
        pltpu.sync_copy(c1_v, o1_hbm.at[pl.ds(base, _PPW)])
        pltpu.sync_copy(c2_v, o2_hbm.at[pl.ds(base, _PPW)])
        pltpu.sync_copy(c3_v, o3_hbm.at[pl.ds(base, _PPW)])


@jax.jit
def _run(x, y, z, table):
    mesh = plsc.VectorSubcoreMesh(core_axis_name="c", subcore_axis_name="s")
    f = functools.partial(
        pl.kernel,
        mesh=mesh,
        out_type=[jax.ShapeDtypeStruct((_N,), jnp.float32)] * 4,
        scratch_types=[
            pltpu.VMEM((_PPW,), jnp.float32),
            pltpu.VMEM((_PPW,), jnp.float32),
            pltpu.VMEM((_PPW,), jnp.float32),
            pltpu.VMEM((_PPW,), jnp.float32),
            pltpu.VMEM((_PPW,), jnp.int32),
            pltpu.VMEM((_PPW,), jnp.int32),
            pltpu.VMEM((_PPW,), jnp.int32),
            pltpu.VMEM((_PPW,), jnp.int32),
            pltpu.VMEM((_PPW,), jnp.float32),
            pltpu.VMEM((_PPW,), jnp.float32),
            pltpu.VMEM((_PPW,), jnp.float32),
            pltpu.VMEM((_PPW,), jnp.float32),
            pltpu.SemaphoreType.DMA((_RING,)),
        ],
        compiler_params=pltpu.CompilerParams(
            needs_layout_passes=False, use_tc_tiling_on_sc=False),
    )(_body)
    return f(x, y, z, table)


def kernel(xyz, d, voxels):
    del d
    table = jnp.transpose(voxels, (0, 1, 3, 2)).reshape(-1)
    c0, c1, c2, c3 = _run(xyz[:, 0], xyz[:, 1], xyz[:, 2], table)
    colors = jnp.stack([c0, c1, c2], axis=1)
    return colors, c3
